# baseline (speedup 1.0000x reference)
import jax
import jax.numpy as jnp
from jax.experimental import pallas as pl
from jax.experimental.pallas import tpu as pltpu


def _make_body(bm, ni, nj):
    def body(x_hbm, w_ref, b_ref, o_ref, xbuf, sem):
        i = pl.program_id(0)
        j = pl.program_id(1)

        def x_copy(blk):
            return pltpu.make_async_copy(
                x_hbm.at[pl.ds(blk * bm, bm), :],
                xbuf.at[blk % 2],
                sem.at[blk % 2],
            )

        @pl.when(j == 0)
        def _prefetch():
            @pl.when(i == 0)
            def _():
                x_copy(0).start()

            @pl.when(i + 1 < ni)
            def _():
                x_copy(i + 1).start()

            x_copy(i).wait()

        acc = jax.lax.dot_general(
            xbuf[i % 2],
            w_ref[...],
            (((1,), (1,)), ((), ())),
            preferred_element_type=jnp.float32,
        )
        o_ref[...] = acc + b_ref[...]

    return body


def kernel(x, W, b):
    m, k = x.shape
    n = W.shape[0]
    bm = min(1024, m)
    bn = min(512, n)
    b2 = b.reshape(1, n)
    ni = m // bm
    nj = n // bn

    def _snake(i, j):
        return jnp.where(i % 2 == 0, j, nj - 1 - j)

    return pl.pallas_call(
        _make_body(bm, ni, nj),
        out_shape=jax.ShapeDtypeStruct((m, n), jnp.float32),
        grid=(ni, nj),
        in_specs=[
            pl.BlockSpec(memory_space=pl.ANY),
            pl.BlockSpec((bn, k), lambda i, j: (_snake(i, j), 0)),
            pl.BlockSpec((1, bn), lambda i, j: (0, _snake(i, j))),
        ],
        out_specs=pl.BlockSpec((bm, bn), lambda i, j: (i, _snake(i, j))),
        scratch_shapes=[
            pltpu.VMEM((2, bm, k), jnp.float32),
            pltpu.SemaphoreType.DMA((2,)),
        ],
        compiler_params=pltpu.CompilerParams(
            dimension_semantics=("arbitrary", "arbitrary"),
        ),
        name="linear_xwt_bias",
    )(x, W, b2)
